# baseline (speedup 1.0000x reference)
import functools

import jax
import jax.numpy as jnp
from jax import lax
from jax.experimental import pallas as pl
from jax.experimental.pallas import tpu as pltpu
from jax.experimental.pallas import tpu_sc as plsc

_LANES = 16


@functools.lru_cache(maxsize=None)
def _make_pool(B, S, Lseq, La, D, V):
    info = plsc.get_sparse_core_info()
    NC, NS = info.num_cores, info.num_subcores
    NW = NC * NS
    N = B * S
    assert N % NW == 0
    SEGS = N // NW
    assert SEGS == _LANES
    assert B == _LANES
    NCH = D // _LANES
    HALF = Lseq // 2
    assert D % _LANES == 0 and Lseq % _LANES == 0 and HALF % 4 == 0
    assert La <= _LANES

    mesh = plsc.VectorSubcoreMesh(core_axis_name="c", subcore_axis_name="s")

    @functools.partial(
        pl.kernel,
        mesh=mesh,
        compiler_params=pltpu.CompilerParams(needs_layout_passes=False),
        out_type=(
            jax.ShapeDtypeStruct((N, D), jnp.float32),
            jax.ShapeDtypeStruct((B, D), jnp.float32),
            jax.ShapeDtypeStruct((N, B), jnp.float32),
            jax.ShapeDtypeStruct((N,), jnp.int32),
            jax.ShapeDtypeStruct((N,), jnp.float32),
            jax.ShapeDtypeStruct((N,), jnp.float32),
        ),
        scratch_types=[
            pltpu.VMEM((SEGS * Lseq,), jnp.int32),
            pltpu.VMEM((HALF, D), jnp.float32),
            pltpu.VMEM((HALF, D), jnp.float32),
            pltpu.VMEM((D,), jnp.float32),
            pltpu.VMEM((D,), jnp.float32),
            pltpu.VMEM((D,), jnp.float32),
            pltpu.VMEM((1, D), jnp.float32),
            pltpu.VMEM((_LANES,), jnp.int32),
            pltpu.VMEM((La, D), jnp.float32),
            pltpu.VMEM((SEGS, B), jnp.float32),
            pltpu.VMEM((SEGS,), jnp.int32),
            pltpu.VMEM((SEGS,), jnp.float32),
            pltpu.VMEM((SEGS,), jnp.float32),
            pltpu.SemaphoreType.DMA,
            pltpu.SemaphoreType.DMA,
            pltpu.SemaphoreType.DMA,
            pltpu.SemaphoreType.DMA,
        ],
    )
    def pool(ts_hbm, asp_hbm, emb_hbm, t_hbm, a_hbm, k_hbm, g_hbm, inv_hbm,
             beta_hbm, toks_v, buf0_v, buf1_v, acc_v, row0_v, row1_v, emb0_v,
             aidx_v, arows_v, kblk_v, gblk_v, invblk_v, betablk_v,
             sem0, sem1, semr0, semr1):
        wid = lax.axis_index("s") * NC + lax.axis_index("c")
        base = wid * SEGS
        lane = lax.iota(jnp.int32, _LANES)

        pltpu.sync_copy(ts_hbm.at[pl.ds(base * Lseq, SEGS * Lseq)], toks_v)
        pltpu.sync_copy(emb_hbm.at[pl.ds(0, 1)], emb0_v)

        def _psum(buf, nrows, sl):
            a0, a1 = buf[0, sl], buf[1, sl]
            a2, a3 = buf[2, sl], buf[3, sl]
            for r in range(4, nrows, 4):
                a0 = a0 + buf[r, sl]
                a1 = a1 + buf[r + 1, sl]
                a2 = a2 + buf[r + 2, sl]
                a3 = a3 + buf[r + 3, sl]
            return (a0 + a1) + (a2 + a3)

        @pl.when(wid < B)
        def _():
            aidx_v[...] = jnp.ones((_LANES,), jnp.int32)
            pltpu.sync_copy(asp_hbm.at[pl.ds(wid * La, La)],
                            aidx_v.at[pl.ds(0, La)])
            pltpu.async_copy(emb_hbm.at[aidx_v.at[pl.ds(0, La)]], arows_v,
                             sem0).wait()
            atok = aidx_v[...]
            n0 = plsc.all_reduce_population_count((atok == 0) & (lane < La))
            n0f = n0.astype(jnp.float32)
            inv = 1.0 / jnp.maximum(La - n0, 1).astype(jnp.float32)
            for c in range(NCH):
                sl = pl.ds(c * _LANES, _LANES)
                tot = _psum(arows_v, La, sl)
                row0_v[sl] = (tot - n0f * emb0_v[0, sl]) * inv
            pltpu.sync_copy(row0_v, a_hbm.at[wid])

        bcol = wid // (S // SEGS)

        def _gather(off, buf, sem):
            return pltpu.async_copy(
                emb_hbm.at[toks_v.at[pl.ds(off, HALF)]], buf, sem)

        def _gwait(off, buf, sem):
            pltpu.make_async_copy(
                emb_hbm.at[toks_v.at[pl.ds(off, HALF)]], buf, sem).wait()

        _gather(0, buf0_v, sem0)

        def one_seg(s, carry, row_v, semr):
            grp_vec, inv_vec, beta_vec = carry
            off = s * Lseq
            _gather(off + HALF, buf1_v, sem1)
            n0 = jnp.zeros((_LANES,), jnp.int32)
            for c in range(Lseq // _LANES):
                tok = toks_v[pl.ds(off + c * _LANES, _LANES)]
                n0 = n0 + plsc.all_reduce_population_count(tok == 0)
            n0f = n0.astype(jnp.float32)
            cnt = Lseq - n0
            inv = 1.0 / jnp.maximum(cnt, 1).astype(jnp.float32)

            _gwait(off, buf0_v, sem0)

            def c_half0(c, _):
                sl = pl.ds(c * _LANES, _LANES)
                acc_v[sl] = _psum(buf0_v, HALF, sl)
                return 0

            lax.fori_loop(0, NCH, c_half0, 0)

            @pl.when(s + 1 < SEGS)
            def _():
                _gather((s + 1) * Lseq, buf0_v, sem0)

            _gwait(off + HALF, buf1_v, sem1)

            @pl.when(s >= 2)
            def _():
                pltpu.make_async_copy(row_v, t_hbm.at[base + s - 2],
                                      semr).wait()

            def c_half1(c, _):
                sl = pl.ds(c * _LANES, _LANES)
                row_v[sl] = acc_v[sl] + _psum(buf1_v, HALF, sl)
                return 0

            lax.fori_loop(0, NCH, c_half1, 0)
            pltpu.async_copy(row_v, t_hbm.at[base + s], semr)

            keep = cnt > 0
            g = jnp.where(keep, bcol, 0)
            kf = jnp.where(keep, 1.0, 0.0)
            kblk_v[s] = jnp.where(lane == bcol, kf, 0.0)
            sel = lane == s
            return (jnp.where(sel, g, grp_vec),
                    jnp.where(sel, inv, inv_vec),
                    jnp.where(sel, n0f * inv, beta_vec))

        def pair_body(p, carry):
            carry = one_seg(p * 2, carry, row0_v, semr0)
            carry = one_seg(p * 2 + 1, carry, row1_v, semr1)
            return carry

        zf = jnp.zeros((_LANES,), jnp.float32)
        grp_vec, inv_vec, beta_vec = lax.fori_loop(
            0, SEGS // 2, pair_body,
            (jnp.zeros((_LANES,), jnp.int32), zf, zf))
        pltpu.make_async_copy(row0_v, t_hbm.at[base + SEGS - 2],
                              semr0).wait()
        pltpu.make_async_copy(row1_v, t_hbm.at[base + SEGS - 1],
                              semr1).wait()

        gblk_v[...] = grp_vec
        invblk_v[...] = inv_vec
        betablk_v[...] = beta_vec
        pltpu.sync_copy(gblk_v, g_hbm.at[pl.ds(base, SEGS)])
        pltpu.sync_copy(invblk_v, inv_hbm.at[pl.ds(base, SEGS)])
        pltpu.sync_copy(betablk_v, beta_hbm.at[pl.ds(base, SEGS)])
        pltpu.sync_copy(kblk_v, k_hbm.at[pl.ds(base, SEGS)])

    return pool


def _head_body(t_ref, a16_ref, k_ref, inv_ref, beta_ref, emb0_ref,
               w1_ref, w2_ref, o_ref):
    f32 = jnp.float32
    D = t_ref.shape[1]
    t = t_ref[...] * inv_ref[...] - beta_ref[...] * emb0_ref[...]
    aw = jnp.dot(a16_ref[...], w1_ref[pl.ds(D, D), :],
                 preferred_element_type=f32)
    h = jnp.tanh(jnp.dot(t, w1_ref[pl.ds(0, D), :],
                         preferred_element_type=f32)
                 + jnp.dot(k_ref[...], aw, preferred_element_type=f32))
    o_ref[...] = jnp.dot(h, w2_ref[...], preferred_element_type=f32)


def kernel(text_slices, aspect_tokens, emb_table, W1, W2):
    B, S, Lseq = text_slices.shape
    La = aspect_tokens.shape[1]
    V, D = emb_table.shape
    N = B * S
    ts = text_slices.reshape(N * Lseq).astype(jnp.int32)
    asp = aspect_tokens.reshape(B * La).astype(jnp.int32)
    emb = emb_table.astype(jnp.float32)
    t, a16, kmat, grp, inv, beta = _make_pool(B, S, Lseq, La, D, V)(
        ts, asp, emb)
    out = pl.pallas_call(
        _head_body,
        out_shape=jax.ShapeDtypeStruct((N, W2.shape[1]), jnp.float32),
    )(t, a16, kmat, inv.reshape(N, 1), beta.reshape(N, 1), emb[:1], W1, W2)
    return out, grp


# speedup vs baseline: 1.0138x; 1.0138x over previous
import functools

import jax
import jax.numpy as jnp
from jax import lax
from jax.experimental import pallas as pl
from jax.experimental.pallas import tpu as pltpu
from jax.experimental.pallas import tpu_sc as plsc

_LANES = 16


@functools.lru_cache(maxsize=None)
def _make_pool(B, S, Lseq, La, D, V):
    info = plsc.get_sparse_core_info()
    NC, NS = info.num_cores, info.num_subcores
    NW = NC * NS
    N = B * S
    assert N % NW == 0
    SEGS = N // NW
    assert SEGS == _LANES
    assert B == _LANES
    NCH = D // _LANES
    HALF = Lseq // 2
    assert D % _LANES == 0 and Lseq % _LANES == 0 and HALF % 4 == 0
    assert La <= _LANES

    mesh = plsc.VectorSubcoreMesh(core_axis_name="c", subcore_axis_name="s")

    @functools.partial(
        pl.kernel,
        mesh=mesh,
        compiler_params=pltpu.CompilerParams(needs_layout_passes=False),
        out_type=(
            jax.ShapeDtypeStruct((N, D), jnp.float32),
            jax.ShapeDtypeStruct((B, D), jnp.float32),
            jax.ShapeDtypeStruct((N, B), jnp.float32),
            jax.ShapeDtypeStruct((N,), jnp.int32),
        ),
        scratch_types=[
            pltpu.VMEM((SEGS * Lseq,), jnp.int32),
            pltpu.VMEM((HALF, D), jnp.float32),
            pltpu.VMEM((HALF, D), jnp.float32),
            pltpu.VMEM((D,), jnp.float32),
            pltpu.VMEM((D,), jnp.float32),
            pltpu.VMEM((D,), jnp.float32),
            pltpu.VMEM((1, D), jnp.float32),
            pltpu.VMEM((_LANES,), jnp.int32),
            pltpu.VMEM((La, D), jnp.float32),
            pltpu.VMEM((SEGS, B), jnp.float32),
            pltpu.VMEM((SEGS,), jnp.int32),
            pltpu.SemaphoreType.DMA,
            pltpu.SemaphoreType.DMA,
            pltpu.SemaphoreType.DMA,
            pltpu.SemaphoreType.DMA,
        ],
    )
    def pool(ts_hbm, asp_hbm, emb_hbm, t_hbm, a_hbm, k_hbm, g_hbm,
             toks_v, buf0_v, buf1_v, acc_v, row0_v, row1_v, emb0_v,
             aidx_v, arows_v, kblk_v, gblk_v, sem0, sem1, semr0, semr1):
        wid = lax.axis_index("s") * NC + lax.axis_index("c")
        base = wid * SEGS
        lane = lax.iota(jnp.int32, _LANES)

        pltpu.sync_copy(ts_hbm.at[pl.ds(base * Lseq, SEGS * Lseq)], toks_v)
        pltpu.sync_copy(emb_hbm.at[pl.ds(0, 1)], emb0_v)

        def _psum(buf, nrows, sl):
            a0, a1 = buf[0, sl], buf[1, sl]
            a2, a3 = buf[2, sl], buf[3, sl]
            for r in range(4, nrows, 4):
                a0 = a0 + buf[r, sl]
                a1 = a1 + buf[r + 1, sl]
                a2 = a2 + buf[r + 2, sl]
                a3 = a3 + buf[r + 3, sl]
            return (a0 + a1) + (a2 + a3)

        @pl.when(wid < B)
        def _():
            aidx_v[...] = jnp.ones((_LANES,), jnp.int32)
            pltpu.sync_copy(asp_hbm.at[pl.ds(wid * La, La)],
                            aidx_v.at[pl.ds(0, La)])
            pltpu.async_copy(emb_hbm.at[aidx_v.at[pl.ds(0, La)]], arows_v,
                             sem0).wait()
            atok = aidx_v[...]
            n0 = plsc.all_reduce_population_count((atok == 0) & (lane < La))
            n0f = n0.astype(jnp.float32)
            inv = 1.0 / jnp.maximum(La - n0, 1).astype(jnp.float32)
            for c in range(NCH):
                sl = pl.ds(c * _LANES, _LANES)
                tot = _psum(arows_v, La, sl)
                row0_v[sl] = (tot - n0f * emb0_v[0, sl]) * inv
            pltpu.sync_copy(row0_v, a_hbm.at[wid])

        bcol = wid // (S // SEGS)

        def _gather(off, buf, sem):
            return pltpu.async_copy(
                emb_hbm.at[toks_v.at[pl.ds(off, HALF)]], buf, sem)

        def _gwait(off, buf, sem):
            pltpu.make_async_copy(
                emb_hbm.at[toks_v.at[pl.ds(off, HALF)]], buf, sem).wait()

        _gather(0, buf0_v, sem0)

        def one_seg(s, grp_vec, row_v, semr):
            off = s * Lseq
            _gather(off + HALF, buf1_v, sem1)
            n0 = jnp.zeros((_LANES,), jnp.int32)
            for c in range(Lseq // _LANES):
                tok = toks_v[pl.ds(off + c * _LANES, _LANES)]
                n0 = n0 + plsc.all_reduce_population_count(tok == 0)
            n0f = n0.astype(jnp.float32)
            cnt = Lseq - n0
            inv = 1.0 / jnp.maximum(cnt, 1).astype(jnp.float32)

            _gwait(off, buf0_v, sem0)

            def c_half0(c, _):
                sl = pl.ds(c * _LANES, _LANES)
                acc_v[sl] = _psum(buf0_v, HALF, sl)
                return 0

            lax.fori_loop(0, NCH, c_half0, 0)

            @pl.when(s + 1 < SEGS)
            def _():
                _gather((s + 1) * Lseq, buf0_v, sem0)

            _gwait(off + HALF, buf1_v, sem1)

            @pl.when(s >= 2)
            def _():
                pltpu.make_async_copy(row_v, t_hbm.at[base + s - 2],
                                      semr).wait()

            def c_half1(c, _):
                sl = pl.ds(c * _LANES, _LANES)
                tot = acc_v[sl] + _psum(buf1_v, HALF, sl)
                row_v[sl] = (tot - n0f * emb0_v[0, sl]) * inv
                return 0

            lax.fori_loop(0, NCH, c_half1, 0)
            pltpu.async_copy(row_v, t_hbm.at[base + s], semr)

            keep = cnt > 0
            g = jnp.where(keep, bcol, 0)
            kf = jnp.where(keep, 1.0, 0.0)
            kblk_v[s] = jnp.where(lane == bcol, kf, 0.0)
            return jnp.where(lane == s, g, grp_vec)

        def pair_body(p, grp_vec):
            grp_vec = one_seg(p * 2, grp_vec, row0_v, semr0)
            grp_vec = one_seg(p * 2 + 1, grp_vec, row1_v, semr1)
            return grp_vec

        grp_vec = lax.fori_loop(0, SEGS // 2, pair_body,
                                jnp.zeros((_LANES,), jnp.int32))
        pltpu.make_async_copy(row0_v, t_hbm.at[base + SEGS - 2],
                              semr0).wait()
        pltpu.make_async_copy(row1_v, t_hbm.at[base + SEGS - 1],
                              semr1).wait()

        gblk_v[...] = grp_vec
        pltpu.sync_copy(gblk_v, g_hbm.at[pl.ds(base, SEGS)])
        pltpu.sync_copy(kblk_v, k_hbm.at[pl.ds(base, SEGS)])

    return pool


def _head_body(t_ref, a16_ref, k_ref, w1_ref, w2_ref, o_ref):
    f32 = jnp.float32
    D = t_ref.shape[1]
    aw = jnp.dot(a16_ref[...], w1_ref[pl.ds(D, D), :],
                 preferred_element_type=f32)
    h = jnp.tanh(jnp.dot(t_ref[...], w1_ref[pl.ds(0, D), :],
                         preferred_element_type=f32)
                 + jnp.dot(k_ref[...], aw, preferred_element_type=f32))
    o_ref[...] = jnp.dot(h, w2_ref[...], preferred_element_type=f32)


def kernel(text_slices, aspect_tokens, emb_table, W1, W2):
    B, S, Lseq = text_slices.shape
    La = aspect_tokens.shape[1]
    V, D = emb_table.shape
    N = B * S
    ts = text_slices.reshape(N * Lseq).astype(jnp.int32)
    asp = aspect_tokens.reshape(B * La).astype(jnp.int32)
    emb = emb_table.astype(jnp.float32)
    t, a16, kmat, grp = _make_pool(B, S, Lseq, La, D, V)(ts, asp, emb)
    out = pl.pallas_call(
        _head_body,
        out_shape=jax.ShapeDtypeStruct((N, W2.shape[1]), jnp.float32),
    )(t, a16, kmat, W1, W2)
    return out, grp


# speedup vs baseline: 1.0758x; 1.0611x over previous
import functools

import jax
import jax.numpy as jnp
from jax import lax
from jax.experimental import pallas as pl
from jax.experimental.pallas import tpu as pltpu
from jax.experimental.pallas import tpu_sc as plsc

_LANES = 16


@functools.lru_cache(maxsize=None)
def _make_pool(B, S, Lseq, La, D, V):
    info = plsc.get_sparse_core_info()
    NC, NS = info.num_cores, info.num_subcores
    NW = NC * NS
    N = B * S
    assert N % NW == 0
    SEGS = N // NW
    assert SEGS == _LANES
    assert B == _LANES
    NCH = D // _LANES
    HALF = Lseq // 2
    assert D % _LANES == 0 and Lseq % _LANES == 0 and HALF % 4 == 0
    assert La <= _LANES

    mesh = plsc.VectorSubcoreMesh(core_axis_name="c", subcore_axis_name="s")

    @functools.partial(
        pl.kernel,
        mesh=mesh,
        compiler_params=pltpu.CompilerParams(needs_layout_passes=False),
        out_type=(
            jax.ShapeDtypeStruct((N, D), jnp.float32),
            jax.ShapeDtypeStruct((B, D), jnp.float32),
            jax.ShapeDtypeStruct((N, B), jnp.float32),
            jax.ShapeDtypeStruct((N,), jnp.int32),
        ),
        scratch_types=[
            pltpu.VMEM((SEGS * Lseq,), jnp.int32),
            pltpu.VMEM((HALF, D), jnp.float32),
            pltpu.VMEM((HALF, D), jnp.float32),
            pltpu.VMEM((D,), jnp.float32),
            pltpu.VMEM((D,), jnp.float32),
            pltpu.VMEM((D,), jnp.float32),
            pltpu.VMEM((1, D), jnp.float32),
            pltpu.VMEM((_LANES,), jnp.int32),
            pltpu.VMEM((La, D), jnp.float32),
            pltpu.VMEM((SEGS, B), jnp.float32),
            pltpu.VMEM((SEGS,), jnp.int32),
            pltpu.SemaphoreType.DMA,
            pltpu.SemaphoreType.DMA,
            pltpu.SemaphoreType.DMA,
            pltpu.SemaphoreType.DMA,
        ],
    )
    def pool(ts_hbm, asp_hbm, emb_hbm, t_hbm, a_hbm, k_hbm, g_hbm,
             toks_v, buf0_v, buf1_v, acc_v, row0_v, row1_v, emb0_v,
             aidx_v, arows_v, kblk_v, gblk_v, sem0, sem1, semr0, semr1):
        wid = lax.axis_index("s") * NC + lax.axis_index("c")
        base = wid * SEGS
        lane = lax.iota(jnp.int32, _LANES)

        pltpu.sync_copy(ts_hbm.at[pl.ds(base * Lseq, SEGS * Lseq)], toks_v)
        pltpu.sync_copy(emb_hbm.at[pl.ds(0, 1)], emb0_v)

        def _psum(buf, nrows, sl):
            a0, a1 = buf[0, sl], buf[1, sl]
            a2, a3 = buf[2, sl], buf[3, sl]
            for r in range(4, nrows, 4):
                a0 = a0 + buf[r, sl]
                a1 = a1 + buf[r + 1, sl]
                a2 = a2 + buf[r + 2, sl]
                a3 = a3 + buf[r + 3, sl]
            return (a0 + a1) + (a2 + a3)

        @pl.when(wid < B)
        def _():
            aidx_v[...] = jnp.ones((_LANES,), jnp.int32)
            pltpu.sync_copy(asp_hbm.at[pl.ds(wid * La, La)],
                            aidx_v.at[pl.ds(0, La)])
            pltpu.async_copy(emb_hbm.at[aidx_v.at[pl.ds(0, La)]], arows_v,
                             sem0).wait()
            atok = aidx_v[...]
            n0 = plsc.all_reduce_population_count((atok == 0) & (lane < La))
            n0f = n0.astype(jnp.float32)
            inv = 1.0 / jnp.maximum(La - n0, 1).astype(jnp.float32)
            for c in range(NCH):
                sl = pl.ds(c * _LANES, _LANES)
                tot = _psum(arows_v, La, sl)
                row0_v[sl] = (tot - n0f * emb0_v[0, sl]) * inv
            pltpu.sync_copy(row0_v, a_hbm.at[wid])

        bcol = wid // (S // SEGS)

        def _gather(off, buf, sem):
            return pltpu.async_copy(
                emb_hbm.at[toks_v.at[pl.ds(off, HALF)]], buf, sem)

        def _gwait(off, buf, sem):
            pltpu.make_async_copy(
                emb_hbm.at[toks_v.at[pl.ds(off, HALF)]], buf, sem).wait()

        _gather(0, buf0_v, sem0)

        def one_seg(s, grp_vec, row_v, semr):
            off = s * Lseq
            _gather(off + HALF, buf1_v, sem1)
            n0 = jnp.zeros((_LANES,), jnp.int32)
            for c in range(Lseq // _LANES):
                tok = toks_v[pl.ds(off + c * _LANES, _LANES)]
                n0 = n0 + plsc.all_reduce_population_count(tok == 0)
            n0f = n0.astype(jnp.float32)
            cnt = Lseq - n0
            inv = 1.0 / jnp.maximum(cnt, 1).astype(jnp.float32)

            _gwait(off, buf0_v, sem0)

            @plsc.parallel_loop(0, NCH, 1)
            def c_half0(c):
                sl = pl.ds(c * _LANES, _LANES)
                acc_v[sl] = _psum(buf0_v, HALF, sl)

            @pl.when(s + 1 < SEGS)
            def _():
                _gather((s + 1) * Lseq, buf0_v, sem0)

            _gwait(off + HALF, buf1_v, sem1)

            @pl.when(s >= 2)
            def _():
                pltpu.make_async_copy(row_v, t_hbm.at[base + s - 2],
                                      semr).wait()

            @plsc.parallel_loop(0, NCH, 1)
            def c_half1(c):
                sl = pl.ds(c * _LANES, _LANES)
                tot = acc_v[sl] + _psum(buf1_v, HALF, sl)
                row_v[sl] = (tot - n0f * emb0_v[0, sl]) * inv
            pltpu.async_copy(row_v, t_hbm.at[base + s], semr)

            keep = cnt > 0
            g = jnp.where(keep, bcol, 0)
            kf = jnp.where(keep, 1.0, 0.0)
            kblk_v[s] = jnp.where(lane == bcol, kf, 0.0)
            return jnp.where(lane == s, g, grp_vec)

        def pair_body(p, grp_vec):
            grp_vec = one_seg(p * 2, grp_vec, row0_v, semr0)
            grp_vec = one_seg(p * 2 + 1, grp_vec, row1_v, semr1)
            return grp_vec

        grp_vec = lax.fori_loop(0, SEGS // 2, pair_body,
                                jnp.zeros((_LANES,), jnp.int32))
        pltpu.make_async_copy(row0_v, t_hbm.at[base + SEGS - 2],
                              semr0).wait()
        pltpu.make_async_copy(row1_v, t_hbm.at[base + SEGS - 1],
                              semr1).wait()

        gblk_v[...] = grp_vec
        pltpu.sync_copy(gblk_v, g_hbm.at[pl.ds(base, SEGS)])
        pltpu.sync_copy(kblk_v, k_hbm.at[pl.ds(base, SEGS)])

    return pool


def _head_body(t_ref, a16_ref, k_ref, w1_ref, w2_ref, o_ref):
    f32 = jnp.float32
    D = t_ref.shape[1]
    aw = jnp.dot(a16_ref[...], w1_ref[pl.ds(D, D), :],
                 preferred_element_type=f32)
    h = jnp.tanh(jnp.dot(t_ref[...], w1_ref[pl.ds(0, D), :],
                         preferred_element_type=f32)
                 + jnp.dot(k_ref[...], aw, preferred_element_type=f32))
    o_ref[...] = jnp.dot(h, w2_ref[...], preferred_element_type=f32)


def kernel(text_slices, aspect_tokens, emb_table, W1, W2):
    B, S, Lseq = text_slices.shape
    La = aspect_tokens.shape[1]
    V, D = emb_table.shape
    N = B * S
    ts = text_slices.reshape(N * Lseq).astype(jnp.int32)
    asp = aspect_tokens.reshape(B * La).astype(jnp.int32)
    emb = emb_table.astype(jnp.float32)
    t, a16, kmat, grp = _make_pool(B, S, Lseq, La, D, V)(ts, asp, emb)
    out = pl.pallas_call(
        _head_body,
        out_shape=jax.ShapeDtypeStruct((N, W2.shape[1]), jnp.float32),
    )(t, a16, kmat, W1, W2)
    return out, grp


# speedup vs baseline: 1.0930x; 1.0160x over previous
import functools

import jax
import jax.numpy as jnp
from jax import lax
from jax.experimental import pallas as pl
from jax.experimental.pallas import tpu as pltpu
from jax.experimental.pallas import tpu_sc as plsc

_LANES = 16


@functools.lru_cache(maxsize=None)
def _make_pool(B, S, Lseq, La, D, V):
    info = plsc.get_sparse_core_info()
    NC, NS = info.num_cores, info.num_subcores
    NW = NC * NS
    N = B * S
    assert N % NW == 0
    SEGS = N // NW
    assert SEGS == _LANES
    assert B == _LANES
    NCH = D // _LANES
    HALF = Lseq // 2
    assert D % _LANES == 0 and Lseq % _LANES == 0 and HALF % 4 == 0
    assert La <= _LANES

    mesh = plsc.VectorSubcoreMesh(core_axis_name="c", subcore_axis_name="s")

    @functools.partial(
        pl.kernel,
        mesh=mesh,
        compiler_params=pltpu.CompilerParams(needs_layout_passes=False),
        out_type=(
            jax.ShapeDtypeStruct((N, D), jnp.float32),
            jax.ShapeDtypeStruct((B, D), jnp.float32),
            jax.ShapeDtypeStruct((N, B), jnp.float32),
            jax.ShapeDtypeStruct((N,), jnp.int32),
        ),
        scratch_types=[
            pltpu.VMEM((SEGS * Lseq,), jnp.int32),
            pltpu.VMEM((HALF, D), jnp.float32),
            pltpu.VMEM((HALF, D), jnp.float32),
            pltpu.VMEM((D,), jnp.float32),
            pltpu.VMEM((D,), jnp.float32),
            pltpu.VMEM((D,), jnp.float32),
            pltpu.VMEM((1, D), jnp.float32),
            pltpu.VMEM((_LANES,), jnp.int32),
            pltpu.VMEM((La, D), jnp.float32),
            pltpu.VMEM((SEGS, B), jnp.float32),
            pltpu.VMEM((SEGS,), jnp.int32),
            pltpu.SemaphoreType.DMA,
            pltpu.SemaphoreType.DMA,
            pltpu.SemaphoreType.DMA,
            pltpu.SemaphoreType.DMA,
        ],
    )
    def pool(ts_hbm, asp_hbm, emb_hbm, t_hbm, a_hbm, k_hbm, g_hbm,
             toks_v, buf0_v, buf1_v, acc_v, row0_v, row1_v, emb0_v,
             aidx_v, arows_v, kblk_v, gblk_v, sem0, sem1, semr0, semr1):
        wid = lax.axis_index("s") * NC + lax.axis_index("c")
        base = wid * SEGS
        lane = lax.iota(jnp.int32, _LANES)

        pltpu.sync_copy(ts_hbm.at[pl.ds(base * Lseq, SEGS * Lseq)], toks_v)
        pltpu.sync_copy(emb_hbm.at[pl.ds(0, 1)], emb0_v)

        def _psum(buf, nrows, sl):
            a0, a1 = buf[0, sl], buf[1, sl]
            a2, a3 = buf[2, sl], buf[3, sl]
            for r in range(4, nrows, 4):
                a0 = a0 + buf[r, sl]
                a1 = a1 + buf[r + 1, sl]
                a2 = a2 + buf[r + 2, sl]
                a3 = a3 + buf[r + 3, sl]
            return (a0 + a1) + (a2 + a3)

        def _aspect():
            aidx_v[...] = jnp.ones((_LANES,), jnp.int32)
            pltpu.sync_copy(asp_hbm.at[pl.ds(wid * La, La)],
                            aidx_v.at[pl.ds(0, La)])
            pltpu.async_copy(emb_hbm.at[aidx_v.at[pl.ds(0, La)]], arows_v,
                             semr1).wait()
            atok = aidx_v[...]
            n0 = plsc.all_reduce_population_count((atok == 0) & (lane < La))
            n0f = n0.astype(jnp.float32)
            inv = 1.0 / jnp.maximum(La - n0, 1).astype(jnp.float32)
            for c in range(NCH):
                sl = pl.ds(c * _LANES, _LANES)
                tot = _psum(arows_v, La, sl)
                row0_v[sl] = (tot - n0f * emb0_v[0, sl]) * inv
            pltpu.sync_copy(row0_v, a_hbm.at[wid])

        bcol = wid // (S // SEGS)

        def _gather(off, buf, sem):
            return pltpu.async_copy(
                emb_hbm.at[toks_v.at[pl.ds(off, HALF)]], buf, sem)

        def _gwait(off, buf, sem):
            pltpu.make_async_copy(
                emb_hbm.at[toks_v.at[pl.ds(off, HALF)]], buf, sem).wait()

        _gather(0, buf0_v, sem0)
        pl.when(wid < B)(_aspect)

        def one_seg(s, grp_vec, row_v, semr):
            off = s * Lseq
            _gather(off + HALF, buf1_v, sem1)
            n0 = jnp.zeros((_LANES,), jnp.int32)
            for c in range(Lseq // _LANES):
                tok = toks_v[pl.ds(off + c * _LANES, _LANES)]
                n0 = n0 + plsc.all_reduce_population_count(tok == 0)
            n0f = n0.astype(jnp.float32)
            cnt = Lseq - n0
            inv = 1.0 / jnp.maximum(cnt, 1).astype(jnp.float32)

            _gwait(off, buf0_v, sem0)

            @plsc.parallel_loop(0, NCH, 1)
            def c_half0(c):
                sl = pl.ds(c * _LANES, _LANES)
                acc_v[sl] = _psum(buf0_v, HALF, sl)

            @pl.when(s + 1 < SEGS)
            def _():
                _gather((s + 1) * Lseq, buf0_v, sem0)

            _gwait(off + HALF, buf1_v, sem1)

            @pl.when(s >= 2)
            def _():
                pltpu.make_async_copy(row_v, t_hbm.at[base + s - 2],
                                      semr).wait()

            @plsc.parallel_loop(0, NCH, 1)
            def c_half1(c):
                sl = pl.ds(c * _LANES, _LANES)
                tot = acc_v[sl] + _psum(buf1_v, HALF, sl)
                row_v[sl] = (tot - n0f * emb0_v[0, sl]) * inv
            pltpu.async_copy(row_v, t_hbm.at[base + s], semr)

            keep = cnt > 0
            g = jnp.where(keep, bcol, 0)
            kf = jnp.where(keep, 1.0, 0.0)
            kblk_v[s] = jnp.where(lane == bcol, kf, 0.0)
            return jnp.where(lane == s, g, grp_vec)

        def pair_body(p, grp_vec):
            grp_vec = one_seg(p * 2, grp_vec, row0_v, semr0)
            grp_vec = one_seg(p * 2 + 1, grp_vec, row1_v, semr1)
            return grp_vec

        grp_vec = lax.fori_loop(0, SEGS // 2, pair_body,
                                jnp.zeros((_LANES,), jnp.int32))
        pltpu.make_async_copy(row0_v, t_hbm.at[base + SEGS - 2],
                              semr0).wait()
        pltpu.make_async_copy(row1_v, t_hbm.at[base + SEGS - 1],
                              semr1).wait()

        gblk_v[...] = grp_vec
        pltpu.sync_copy(gblk_v, g_hbm.at[pl.ds(base, SEGS)])
        pltpu.sync_copy(kblk_v, k_hbm.at[pl.ds(base, SEGS)])

    return pool


def _head_body(t_ref, a16_ref, k_ref, w1_ref, w2_ref, o_ref):
    f32 = jnp.float32
    D = t_ref.shape[1]
    aw = jnp.dot(a16_ref[...], w1_ref[pl.ds(D, D), :],
                 preferred_element_type=f32)
    h = jnp.tanh(jnp.dot(t_ref[...], w1_ref[pl.ds(0, D), :],
                         preferred_element_type=f32)
                 + jnp.dot(k_ref[...], aw, preferred_element_type=f32))
    o_ref[...] = jnp.dot(h, w2_ref[...], preferred_element_type=f32)


def kernel(text_slices, aspect_tokens, emb_table, W1, W2):
    B, S, Lseq = text_slices.shape
    La = aspect_tokens.shape[1]
    V, D = emb_table.shape
    N = B * S
    ts = text_slices.reshape(N * Lseq).astype(jnp.int32)
    asp = aspect_tokens.reshape(B * La).astype(jnp.int32)
    emb = emb_table.astype(jnp.float32)
    t, a16, kmat, grp = _make_pool(B, S, Lseq, La, D, V)(ts, asp, emb)
    out = pl.pallas_call(
        _head_body,
        out_shape=jax.ShapeDtypeStruct((N, W2.shape[1]), jnp.float32),
    )(t, a16, kmat, W1, W2)
    return out, grp
